# baseline (speedup 1.0000x reference)
import functools

import jax
import jax.numpy as jnp
from jax import lax
from jax.experimental import pallas as pl
from jax.experimental.pallas import tpu as pltpu
from jax.experimental.pallas import tpu_sc as plsc

N = 50000
E = 1600000
T = 10

NTILE = 16
NPAD = 50176
NPT = NPAD // NTILE
EPW = E // NTILE
CH = 2000
NCH = EPW // CH
TABN = 256
HW = (T + 1) * 16


def _sir_body(ei_hbm, g_hbm, thr_hbm,
              consts_hbm, hi_hbm, hr_hbm, inf_hbm, part_hbm, ep_hbm,
              inf_v, cnt_v, sA, sB, dA, dB, tmpA, tmpB, sus_v,
              gi_v, gr_v, thr_v, consts_v, hi_v, hr_v,
              sem_inf, semA, semB, sem_di, sem_dr):
    tid = lax.axis_index("s")
    sb = tid * NPT
    eb = tid * EPW

    pltpu.sync_copy(thr_hbm, thr_v)
    pltpu.sync_copy(consts_hbm, consts_v)
    c0v = consts_v[pl.ds(0, 16)]
    crv = consts_v[pl.ds(16, 16)]
    lanes = lax.iota(jnp.int32, 16)
    lanes2 = lanes * 2

    pltpu.sync_copy(g_hbm.at[pl.ds(2 * sb, 2 * NPT)], gi_v)

    @plsc.parallel_loop(0, NPT, 16, unroll=4,
                        carry=jnp.zeros((16,), jnp.float32))
    def si0(i, s):
        idx0 = lanes2 + 2 * i
        g0 = plsc.load_gather(gi_v, [idx0])
        g1 = plsc.load_gather(gi_v, [idx0 + 1])
        g = g0 - g1
        idxv = lanes + (sb + i)
        valid = jnp.where(idxv < N, 1.0, 0.0).astype(jnp.float32)
        infv = valid * jnp.where(g > c0v, 1.0, 0.0).astype(jnp.float32)
        tmpA[pl.ds(i, 16)] = infv
        sus_v[pl.ds(i, 16)] = valid - infv
        return s + infv

    hi_v[pl.ds(0, 16)] = si0
    hr_v[pl.ds(0, 16)] = jnp.zeros((16,), jnp.float32)
    pltpu.sync_copy(tmpA, inf_hbm.at[pl.ds(sb, NPT)])
    plsc.subcore_barrier()

    def step(t, carry):
        def stage_phase():
            cp_inf = pltpu.async_copy(inf_hbm, inf_v, sem_inf)
            pltpu.async_copy(
                g_hbm.at[pl.ds((1 + t) * 2 * N + 2 * sb, 2 * NPT)], gi_v,
                sem_di)
            pltpu.async_copy(
                g_hbm.at[pl.ds((1 + T + t) * 2 * N + 2 * sb, 2 * NPT)],
                gr_v, sem_dr)

            @plsc.parallel_loop(0, NPAD, 16, unroll=8)
            def _zero(i):
                cnt_v[pl.ds(i, 16)] = jnp.zeros((16,), jnp.float32)

            cp_inf.wait()

        U = 25

        def process(sbuf, dbuf):
            @plsc.parallel_loop(0, CH, 16 * U)
            def _edges(i):
                sidx = [sbuf[pl.ds(i + 16 * k, 16)] for k in range(U)]
                didx = [dbuf[pl.ds(i + 16 * k, 16)] for k in range(U)]
                vals = [plsc.load_gather(inf_v, [s]) for s in sidx]
                for d, v in zip(didx, vals):
                    plsc.addupdate_scatter(cnt_v, [d], v)

        def start_chunk(c, sbuf, dbuf, sem):
            pltpu.async_copy(ei_hbm.at[0, pl.ds(eb + c * CH, CH)], sbuf,
                             sem)
            pltpu.async_copy(ei_hbm.at[1, pl.ds(eb + c * CH, CH)], dbuf,
                             sem)

        def wait_chunk(sbuf, dbuf, sem):
            pltpu.make_async_copy(
                ei_hbm.at[0, pl.ds(eb, CH)], sbuf, sem).wait()
            pltpu.make_async_copy(
                ei_hbm.at[1, pl.ds(eb, CH)], dbuf, sem).wait()

        def edges_phase_pack():
            def chunk(c, c2):
                pltpu.sync_copy(ei_hbm.at[0, pl.ds(eb + c * CH, CH)], sA)
                pltpu.sync_copy(ei_hbm.at[1, pl.ds(eb + c * CH, CH)], dA)
                process(sA, dA)

                @plsc.parallel_loop(0, CH, 16, unroll=8)
                def _pack(i):
                    s = sA[pl.ds(i, 16)]
                    d = dA[pl.ds(i, 16)]
                    sA[pl.ds(i, 16)] = s + lax.shift_left(d, 16)

                pltpu.sync_copy(sA, ep_hbm.at[pl.ds(eb + c * CH, CH)])
                return c2

            lax.fori_loop(0, NCH, chunk, 0)

        def process_packed(buf):
            @plsc.parallel_loop(0, CH, 16 * U)
            def _edges(i):
                ws = [buf[pl.ds(i + 16 * k, 16)] for k in range(U)]
                sidx = [lax.bitwise_and(w, 0xFFFF) for w in ws]
                didx = [lax.shift_right_logical(w, 16) for w in ws]
                vals = [plsc.load_gather(inf_v, [s]) for s in sidx]
                for d, v in zip(didx, vals):
                    plsc.addupdate_scatter(cnt_v, [d], v)

        def start_pchunk(c, buf, sem):
            pltpu.async_copy(ep_hbm.at[pl.ds(eb + c * CH, CH)], buf, sem)

        def wait_pchunk(buf, sem):
            pltpu.make_async_copy(
                ep_hbm.at[pl.ds(eb, CH)], buf, sem).wait()

        def edges_phase_packed():
            start_pchunk(0, sA, semA)

            def pair(j, c2):
                wait_pchunk(sA, semA)
                start_pchunk(2 * j + 1, sB, semB)
                process_packed(sA)
                wait_pchunk(sB, semB)

                @pl.when(j + 1 < NCH // 2)
                def _():
                    start_pchunk(2 * j + 2, sA, semA)

                process_packed(sB)
                return c2

            lax.fori_loop(0, NCH // 2, pair, 0)

        def edges_phase():
            @pl.when(t == 0)
            def _():
                edges_phase_pack()

            @pl.when(t > 0)
            def _():
                edges_phase_packed()

        def reduce_phase():
            pltpu.sync_copy(cnt_v, part_hbm.at[pl.ds(tid * NPAD, NPAD)])
            plsc.subcore_barrier()

            pltpu.async_copy(
                part_hbm.at[pl.ds(0 * NPAD + sb, NPT)], tmpA, semA)
            for k in range(NTILE):
                cur, csem = (tmpA, semA) if k % 2 == 0 else (tmpB, semB)
                pltpu.make_async_copy(
                    part_hbm.at[pl.ds(sb, NPT)], cur, csem).wait()
                if k + 1 < NTILE:
                    nxt, nsem = (tmpB, semB) if k % 2 == 0 else (tmpA, semA)
                    pltpu.async_copy(
                        part_hbm.at[pl.ds((k + 1) * NPAD + sb, NPT)], nxt,
                        nsem)
                if k == 0:
                    @plsc.parallel_loop(0, NPT, 16, unroll=8)
                    def _racc(i):
                        cnt_v[pl.ds(sb + i, 16)] = cur[pl.ds(i, 16)]
                else:
                    @plsc.parallel_loop(0, NPT, 16, unroll=8)
                    def _racc(i):
                        cnt_v[pl.ds(sb + i, 16)] = (
                            cnt_v[pl.ds(sb + i, 16)] + cur[pl.ds(i, 16)])

        def update_phase():
            pltpu.make_async_copy(
                g_hbm.at[pl.ds(0, 2 * NPT)], gi_v, sem_di).wait()
            pltpu.make_async_copy(
                g_hbm.at[pl.ds(0, 2 * NPT)], gr_v, sem_dr).wait()

            @plsc.parallel_loop(0, NPT, 16, unroll=4,
                                carry=(jnp.zeros((16,), jnp.float32),
                                       jnp.zeros((16,), jnp.float32)))
            def sums(i, c3):
                si, sr = c3
                idx0 = lanes2 + 2 * i
                di = (plsc.load_gather(gi_v, [idx0])
                      - plsc.load_gather(gi_v, [idx0 + 1]))
                dr = (plsc.load_gather(gr_v, [idx0])
                      - plsc.load_gather(gr_v, [idx0 + 1]))
                cntv = cnt_v[pl.ds(sb + i, 16)]
                susv = sus_v[pl.ds(i, 16)]
                infv = inf_v[pl.ds(sb + i, 16)]
                nm = jnp.minimum(cntv * susv, float(TABN - 1))
                thr = plsc.load_gather(thr_v, [nm.astype(jnp.int32)])
                newi = jnp.where(di > thr, 1.0, 0.0).astype(jnp.float32)
                newr = infv * jnp.where(dr > crv, 1.0, 0.0
                                        ).astype(jnp.float32)
                inf2 = infv + newi - newr
                sus_v[pl.ds(i, 16)] = susv - newi
                tmpA[pl.ds(i, 16)] = inf2
                return (si + inf2, sr + newr)

            si, sr = sums
            hi_v[pl.ds(t * 16 + 16, 16)] = si
            hr_v[pl.ds(t * 16 + 16, 16)] = sr
            pltpu.sync_copy(tmpA, inf_hbm.at[pl.ds(sb, NPT)])
            plsc.subcore_barrier()

        with jax.named_scope("ph_stage"):
            stage_phase()
        with jax.named_scope("ph_edges"):
            edges_phase()
        with jax.named_scope("ph_reduce"):
            reduce_phase()
        with jax.named_scope("ph_update"):
            update_phase()
        return carry

    lax.fori_loop(0, T, step, 0)
    pltpu.sync_copy(hi_v, hi_hbm.at[tid])
    pltpu.sync_copy(hr_v, hr_hbm.at[tid])


_sir_call = functools.partial(
    pl.kernel,
    mesh=plsc.VectorSubcoreMesh(
        core_axis_name="c", subcore_axis_name="s", num_cores=1),
    compiler_params=pltpu.CompilerParams(
        needs_layout_passes=False, use_tc_tiling_on_sc=False),
    out_type=(
        jax.ShapeDtypeStruct((NTILE, HW), jnp.float32),
        jax.ShapeDtypeStruct((NTILE, HW), jnp.float32),
        jax.ShapeDtypeStruct((NPAD,), jnp.float32),
        jax.ShapeDtypeStruct((NTILE * NPAD,), jnp.float32),
        jax.ShapeDtypeStruct((E,), jnp.int32),
    ),
    scratch_types=[
        pltpu.VMEM((NPAD,), jnp.float32),
        pltpu.VMEM((NPAD,), jnp.float32),
        pltpu.VMEM((CH,), jnp.int32),
        pltpu.VMEM((CH,), jnp.int32),
        pltpu.VMEM((CH,), jnp.int32),
        pltpu.VMEM((CH,), jnp.int32),
        pltpu.VMEM((NPT,), jnp.float32),
        pltpu.VMEM((NPT,), jnp.float32),
        pltpu.VMEM((NPT,), jnp.float32),
        pltpu.VMEM((2 * NPT,), jnp.float32),
        pltpu.VMEM((2 * NPT,), jnp.float32),
        pltpu.VMEM((TABN,), jnp.float32),
        pltpu.VMEM((32,), jnp.float32),
        pltpu.VMEM((HW,), jnp.float32),
        pltpu.VMEM((HW,), jnp.float32),
        pltpu.SemaphoreType.DMA,
        pltpu.SemaphoreType.DMA,
        pltpu.SemaphoreType.DMA,
        pltpu.SemaphoreType.DMA,
        pltpu.SemaphoreType.DMA,
    ],
)(_sir_body)


def kernel(params, edge_index):
    dtype = params.dtype
    p0, beta, gamma = params[0], params[1], params[2]
    base = jax.random.key(12345)

    nkeys = [0] + [3 * t + 1 for t in range(T)] + [3 * t + 2 for t in range(T)]
    kv = jax.vmap(lambda i: jax.random.fold_in(base, i))(jnp.array(nkeys))
    u = jax.vmap(lambda k: jax.random.uniform(
        k, (2 * N,), minval=1e-6, maxval=1.0 - 1e-6))(kv)
    g = -jnp.log(-jnp.log(u))
    g_flat = jnp.pad(g.reshape(-1), (0, 2 * (NPAD - N) + 16))

    c0 = jnp.log(1.0 - p0) - jnp.log(p0)
    log1mb = jnp.log(1.0 - beta)
    c_rec = jnp.log(1.0 - gamma) - jnp.log(gamma)
    consts = jnp.concatenate([jnp.full((16,), c0, jnp.float32),
                              jnp.full((16,), c_rec, jnp.float32)])
    a = jnp.arange(TABN, dtype=jnp.float32) * log1mb
    thr_tab = a - jnp.log(1.0 - jnp.exp(a))

    hi, hr, _, _, _ = _sir_call(edge_index.astype(jnp.int32), g_flat,
                                thr_tab, consts)

    infected_hist = hi.reshape(NTILE, T + 1, 16).sum(axis=(0, 2)).astype(dtype)
    rec_deltas = hr.reshape(NTILE, T + 1, 16).sum(axis=(0, 2))
    recovered_hist = jnp.cumsum(rec_deltas).astype(dtype)
    return (infected_hist, recovered_hist)


# speedup vs baseline: 1.0123x; 1.0123x over previous
import functools

import jax
import jax.numpy as jnp
from jax import lax
from jax.experimental import pallas as pl
from jax.experimental.pallas import tpu as pltpu
from jax.experimental.pallas import tpu_sc as plsc

N = 50000
E = 1600000
T = 10

NTILE = 16
NPAD = 50176
NPT = NPAD // NTILE
EPW = E // NTILE
CH = 2000
NCH = EPW // CH
TABN = 256
HW = (T + 1) * 16


def _sir_body(ei_hbm, g_hbm, thr_hbm,
              consts_hbm, hi_hbm, hr_hbm, inf_hbm, part_hbm,
              inf_v, cnt_v, sA, sB, dA, dB, tmpA, tmpB, sus_v,
              gi_v, gr_v, thr_v, consts_v, hi_v, hr_v,
              sem_inf, semA, semB, sem_di, sem_dr):
    tid = lax.axis_index("s")
    sb = tid * NPT
    eb = tid * EPW

    pltpu.sync_copy(thr_hbm, thr_v)
    pltpu.sync_copy(consts_hbm, consts_v)
    c0v = consts_v[pl.ds(0, 16)]
    crv = consts_v[pl.ds(16, 16)]
    lanes = lax.iota(jnp.int32, 16)
    lanes2 = lanes * 2

    pltpu.sync_copy(g_hbm.at[pl.ds(2 * sb, 2 * NPT)], gi_v)

    @plsc.parallel_loop(0, NPT, 16, unroll=4,
                        carry=jnp.zeros((16,), jnp.float32))
    def si0(i, s):
        idx0 = lanes2 + 2 * i
        g0 = plsc.load_gather(gi_v, [idx0])
        g1 = plsc.load_gather(gi_v, [idx0 + 1])
        g = g0 - g1
        idxv = lanes + (sb + i)
        valid = jnp.where(idxv < N, 1.0, 0.0).astype(jnp.float32)
        infv = valid * jnp.where(g > c0v, 1.0, 0.0).astype(jnp.float32)
        tmpA[pl.ds(i, 16)] = infv
        sus_v[pl.ds(i, 16)] = valid - infv
        return s + infv

    hi_v[pl.ds(0, 16)] = si0
    hr_v[pl.ds(0, 16)] = jnp.zeros((16,), jnp.float32)
    pltpu.sync_copy(tmpA, inf_hbm.at[pl.ds(sb, NPT)])
    plsc.subcore_barrier()

    def step(t, carry):
        def stage_phase():
            cp_inf = pltpu.async_copy(inf_hbm, inf_v, sem_inf)
            pltpu.async_copy(
                g_hbm.at[pl.ds((1 + t) * 2 * N + 2 * sb, 2 * NPT)], gi_v,
                sem_di)
            pltpu.async_copy(
                g_hbm.at[pl.ds((1 + T + t) * 2 * N + 2 * sb, 2 * NPT)],
                gr_v, sem_dr)

            @plsc.parallel_loop(0, NPAD, 16, unroll=8)
            def _zero(i):
                cnt_v[pl.ds(i, 16)] = jnp.zeros((16,), jnp.float32)

            cp_inf.wait()

        U = 25

        def process(sbuf, dbuf):
            @plsc.parallel_loop(0, CH, 16 * U)
            def _edges(i):
                sidx = [sbuf[pl.ds(i + 16 * k, 16)] for k in range(U)]
                didx = [dbuf[pl.ds(i + 16 * k, 16)] for k in range(U)]
                vals = [plsc.load_gather(inf_v, [s]) for s in sidx]
                for d, v in zip(didx, vals):
                    plsc.addupdate_scatter(cnt_v, [d], v)

        def start_chunk(c, sbuf, dbuf, sem):
            pltpu.async_copy(ei_hbm.at[0, pl.ds(eb + c * CH, CH)], sbuf,
                             sem)
            pltpu.async_copy(ei_hbm.at[1, pl.ds(eb + c * CH, CH)], dbuf,
                             sem)

        def wait_chunk(sbuf, dbuf, sem):
            pltpu.make_async_copy(
                ei_hbm.at[0, pl.ds(eb, CH)], sbuf, sem).wait()
            pltpu.make_async_copy(
                ei_hbm.at[1, pl.ds(eb, CH)], dbuf, sem).wait()

        def edges_phase():
            start_chunk(0, sA, dA, semA)

            def pair(j, c2):
                wait_chunk(sA, dA, semA)
                start_chunk(2 * j + 1, sB, dB, semB)
                process(sA, dA)
                wait_chunk(sB, dB, semB)

                @pl.when(j + 1 < NCH // 2)
                def _():
                    start_chunk(2 * j + 2, sA, dA, semA)

                process(sB, dB)
                return c2

            lax.fori_loop(0, NCH // 2, pair, 0)

        def reduce_phase():
            pltpu.sync_copy(cnt_v, part_hbm.at[pl.ds(tid * NPAD, NPAD)])
            plsc.subcore_barrier()

            pltpu.async_copy(
                part_hbm.at[pl.ds(0 * NPAD + sb, NPT)], tmpA, semA)
            for k in range(NTILE):
                cur, csem = (tmpA, semA) if k % 2 == 0 else (tmpB, semB)
                pltpu.make_async_copy(
                    part_hbm.at[pl.ds(sb, NPT)], cur, csem).wait()
                if k + 1 < NTILE:
                    nxt, nsem = (tmpB, semB) if k % 2 == 0 else (tmpA, semA)
                    pltpu.async_copy(
                        part_hbm.at[pl.ds((k + 1) * NPAD + sb, NPT)], nxt,
                        nsem)
                if k == 0:
                    @plsc.parallel_loop(0, NPT, 16, unroll=8)
                    def _racc(i):
                        cnt_v[pl.ds(sb + i, 16)] = cur[pl.ds(i, 16)]
                else:
                    @plsc.parallel_loop(0, NPT, 16, unroll=8)
                    def _racc(i):
                        cnt_v[pl.ds(sb + i, 16)] = (
                            cnt_v[pl.ds(sb + i, 16)] + cur[pl.ds(i, 16)])

        def update_phase():
            pltpu.make_async_copy(
                g_hbm.at[pl.ds(0, 2 * NPT)], gi_v, sem_di).wait()
            pltpu.make_async_copy(
                g_hbm.at[pl.ds(0, 2 * NPT)], gr_v, sem_dr).wait()

            @plsc.parallel_loop(0, NPT, 16, unroll=4,
                                carry=(jnp.zeros((16,), jnp.float32),
                                       jnp.zeros((16,), jnp.float32)))
            def sums(i, c3):
                si, sr = c3
                idx0 = lanes2 + 2 * i
                di = (plsc.load_gather(gi_v, [idx0])
                      - plsc.load_gather(gi_v, [idx0 + 1]))
                dr = (plsc.load_gather(gr_v, [idx0])
                      - plsc.load_gather(gr_v, [idx0 + 1]))
                cntv = cnt_v[pl.ds(sb + i, 16)]
                susv = sus_v[pl.ds(i, 16)]
                infv = inf_v[pl.ds(sb + i, 16)]
                nm = jnp.minimum(cntv * susv, float(TABN - 1))
                thr = plsc.load_gather(thr_v, [nm.astype(jnp.int32)])
                newi = jnp.where(di > thr, 1.0, 0.0).astype(jnp.float32)
                newr = infv * jnp.where(dr > crv, 1.0, 0.0
                                        ).astype(jnp.float32)
                inf2 = infv + newi - newr
                sus_v[pl.ds(i, 16)] = susv - newi
                tmpA[pl.ds(i, 16)] = inf2
                return (si + inf2, sr + newr)

            si, sr = sums
            hi_v[pl.ds(t * 16 + 16, 16)] = si
            hr_v[pl.ds(t * 16 + 16, 16)] = sr
            pltpu.sync_copy(tmpA, inf_hbm.at[pl.ds(sb, NPT)])
            plsc.subcore_barrier()

        with jax.named_scope("ph_stage"):
            stage_phase()
        with jax.named_scope("ph_edges"):
            edges_phase()
        with jax.named_scope("ph_reduce"):
            reduce_phase()
        with jax.named_scope("ph_update"):
            update_phase()
        return carry

    lax.fori_loop(0, T, step, 0)
    pltpu.sync_copy(hi_v, hi_hbm.at[tid])
    pltpu.sync_copy(hr_v, hr_hbm.at[tid])


_sir_call = functools.partial(
    pl.kernel,
    mesh=plsc.VectorSubcoreMesh(
        core_axis_name="c", subcore_axis_name="s", num_cores=1),
    compiler_params=pltpu.CompilerParams(
        needs_layout_passes=False, use_tc_tiling_on_sc=False),
    out_type=(
        jax.ShapeDtypeStruct((NTILE, HW), jnp.float32),
        jax.ShapeDtypeStruct((NTILE, HW), jnp.float32),
        jax.ShapeDtypeStruct((NPAD,), jnp.float32),
        jax.ShapeDtypeStruct((NTILE * NPAD,), jnp.float32),
    ),
    scratch_types=[
        pltpu.VMEM((NPAD,), jnp.float32),
        pltpu.VMEM((NPAD,), jnp.float32),
        pltpu.VMEM((CH,), jnp.int32),
        pltpu.VMEM((CH,), jnp.int32),
        pltpu.VMEM((CH,), jnp.int32),
        pltpu.VMEM((CH,), jnp.int32),
        pltpu.VMEM((NPT,), jnp.float32),
        pltpu.VMEM((NPT,), jnp.float32),
        pltpu.VMEM((NPT,), jnp.float32),
        pltpu.VMEM((2 * NPT,), jnp.float32),
        pltpu.VMEM((2 * NPT,), jnp.float32),
        pltpu.VMEM((TABN,), jnp.float32),
        pltpu.VMEM((32,), jnp.float32),
        pltpu.VMEM((HW,), jnp.float32),
        pltpu.VMEM((HW,), jnp.float32),
        pltpu.SemaphoreType.DMA,
        pltpu.SemaphoreType.DMA,
        pltpu.SemaphoreType.DMA,
        pltpu.SemaphoreType.DMA,
        pltpu.SemaphoreType.DMA,
    ],
)(_sir_body)


def kernel(params, edge_index):
    dtype = params.dtype
    p0, beta, gamma = params[0], params[1], params[2]
    base = jax.random.key(12345)

    nkeys = [0] + [3 * t + 1 for t in range(T)] + [3 * t + 2 for t in range(T)]
    kv = jax.vmap(lambda i: jax.random.fold_in(base, i))(jnp.array(nkeys))
    u = jax.vmap(lambda k: jax.random.uniform(
        k, (2 * N,), minval=1e-6, maxval=1.0 - 1e-6))(kv)
    g = -jnp.log(-jnp.log(u))
    g_flat = jnp.pad(g.reshape(-1), (0, 2 * (NPAD - N) + 16))

    c0 = jnp.log(1.0 - p0) - jnp.log(p0)
    log1mb = jnp.log(1.0 - beta)
    c_rec = jnp.log(1.0 - gamma) - jnp.log(gamma)
    consts = jnp.concatenate([jnp.full((16,), c0, jnp.float32),
                              jnp.full((16,), c_rec, jnp.float32)])
    a = jnp.arange(TABN, dtype=jnp.float32) * log1mb
    thr_tab = a - jnp.log(1.0 - jnp.exp(a))

    hi, hr, _, _ = _sir_call(edge_index.astype(jnp.int32), g_flat, thr_tab,
                             consts)

    infected_hist = hi.reshape(NTILE, T + 1, 16).sum(axis=(0, 2)).astype(dtype)
    rec_deltas = hr.reshape(NTILE, T + 1, 16).sum(axis=(0, 2))
    recovered_hist = jnp.cumsum(rec_deltas).astype(dtype)
    return (infected_hist, recovered_hist)
